# baseline (speedup 1.0000x reference)
import functools

import jax
import jax.numpy as jnp
from jax import lax
from jax.experimental import pallas as pl
from jax.experimental.pallas import tpu as pltpu
from jax.experimental.pallas import tpu_sc as plsc

try:
    _info = plsc.get_sparse_core_info()
    _NC, _NS = _info.num_cores, _info.num_subcores
except Exception:
    _NC, _NS = 2, 16
_NW = _NC * _NS

_CH = 64


@functools.partial(jax.jit, static_argnums=(2,))
def _gather_rows(x, nlf, nb):
    tbl_rows, xdim = x.shape
    n_per_batch = tbl_rows // nb
    _, nch, ch = nlf.shape
    rows = nch * ch
    nb_per_core = nb // _NC

    mesh = plsc.VectorSubcoreMesh(core_axis_name="c", subcore_axis_name="s")

    @functools.partial(
        pl.kernel,
        mesh=mesh,
        out_type=jax.ShapeDtypeStruct((_NW * rows, xdim), x.dtype),
        scratch_types=[
            pltpu.VMEM_SHARED((nb_per_core, n_per_batch, xdim), x.dtype),
            pltpu.VMEM((nch, ch), jnp.int32),
            pltpu.VMEM((ch, xdim), x.dtype),
            pltpu.VMEM((ch, xdim), x.dtype),
            pltpu.VMEM((ch, xdim), x.dtype),
            pltpu.VMEM((ch, xdim), x.dtype),
            pltpu.SemaphoreType.DMA,
            pltpu.SemaphoreType.DMA,
        ],
    )
    def k(x_hbm, nl_hbm, out_hbm, tbl_sh, idx_v, buf0, buf1, buf2, buf3,
          gsem, ssem):
        bufs = (buf0, buf1, buf2, buf3)
        cid = lax.axis_index("c")
        sid = lax.axis_index("s")
        wid = cid * _NS + sid
        base = wid * rows

        tiles_per_batch = _NS // nb_per_core
        j = sid // tiles_per_batch
        h = sid - j * tiles_per_batch
        slice_rows = n_per_batch // tiles_per_batch
        pltpu.sync_copy(
            x_hbm.at[pl.ds((cid * nb_per_core + j) * n_per_batch
                           + h * slice_rows, slice_rows)],
            tbl_sh.at[j].at[pl.ds(h * slice_rows, slice_rows)])
        pltpu.sync_copy(nl_hbm.at[wid], idx_v)
        plsc.subcore_barrier()

        xb = tbl_sh.at[j]

        def gather(c, buf):
            return pltpu.make_async_copy(xb.at[idx_v.at[c]], buf, gsem)

        def scatter(c, buf):
            return pltpu.make_async_copy(
                buf, out_hbm.at[pl.ds(base + c * ch, ch)], ssem)

        gather(0, bufs[0]).start()
        gather(1, bufs[1]).start()
        for d in range(4):
            gather(d, bufs[d % 4]).wait()
            scatter(d, bufs[d % 4]).start()
            if d >= 2:
                scatter(d - 2, bufs[(d - 2) % 4]).wait()
            gather(d + 2, bufs[(d + 2) % 4]).start()

        @pl.loop(4, nch - 8, step=4)
        def _(c):
            for k in range(4):
                d = c + k
                gather(d, bufs[k]).wait()
                scatter(d, bufs[k]).start()
                scatter(d - 2, bufs[(k + 2) % 4]).wait()
                gather(d + 2, bufs[(k + 2) % 4]).start()

        for dd in range(nch - 8, nch):
            gather(dd, bufs[dd % 4]).wait()
            scatter(dd, bufs[dd % 4]).start()
            scatter(dd - 2, bufs[(dd - 2) % 4]).wait()
            if dd + 2 < nch:
                gather(dd + 2, bufs[(dd + 2) % 4]).start()

        scatter(nch - 2, bufs[(nch - 2) % 4]).wait()
        scatter(nch - 1, bufs[(nch - 1) % 4]).wait()

    return k(x, nlf)


def kernel(x, neighbor_list):
    b, n, xdim = x.shape
    l = neighbor_list.shape[-1]
    rows_total = b * n * l
    rows_per_w = rows_total // _NW
    assert rows_total % _NW == 0 and rows_per_w % _CH == 0
    assert (n * l) % rows_per_w == 0

    nlf = neighbor_list.reshape(_NW, rows_per_w // _CH, _CH)
    out = _gather_rows(x.reshape(b * n, xdim), nlf, b)
    return out.reshape(b, n, l, xdim)


# speedup vs baseline: 1.0326x; 1.0326x over previous
import functools

import jax
import jax.numpy as jnp
from jax import lax
from jax.experimental import pallas as pl
from jax.experimental.pallas import tpu as pltpu
from jax.experimental.pallas import tpu_sc as plsc

try:
    _info = plsc.get_sparse_core_info()
    _NC, _NS = _info.num_cores, _info.num_subcores
except Exception:
    _NC, _NS = 2, 16
_NW = _NC * _NS

_CH = 32


@functools.partial(jax.jit, static_argnums=(2,))
def _gather_rows(x, nlf, nb):
    tbl_rows, xdim = x.shape
    n_per_batch = tbl_rows // nb
    _, nch, ch = nlf.shape
    rows = nch * ch
    npass = nb // _NC
    slice_rows = n_per_batch // _NS

    mesh = plsc.VectorSubcoreMesh(core_axis_name="c", subcore_axis_name="s")

    @functools.partial(
        pl.kernel,
        mesh=mesh,
        out_type=jax.ShapeDtypeStruct((nb * _NS * rows, xdim), x.dtype),
        scratch_types=[
            pltpu.VMEM_SHARED((n_per_batch, xdim), x.dtype),
            pltpu.VMEM((nch, ch), jnp.int32),
        ] + [pltpu.VMEM((ch, xdim), x.dtype)] * 8 + [
            pltpu.SemaphoreType.DMA,
            pltpu.SemaphoreType.DMA,
        ],
    )
    def k(x_hbm, nl_hbm, out_hbm, tbl_sh, idx_v, b0, b1, b2, b3, b4, b5,
          b6, b7, gsem, ssem):
        bufs = (b0, b1, b2, b3, b4, b5, b6, b7)
        cid = lax.axis_index("c")
        sid = lax.axis_index("s")

        for p in range(npass):
            batch = p * _NC + cid

            if p > 0:
                plsc.subcore_barrier()

            pltpu.sync_copy(
                x_hbm.at[pl.ds(batch * n_per_batch + sid * slice_rows,
                               slice_rows)],
                tbl_sh.at[pl.ds(sid * slice_rows, slice_rows)])
            pltpu.sync_copy(nl_hbm.at[batch * _NS + sid], idx_v)
            plsc.subcore_barrier()

            base = (batch * _NS + sid) * rows

            def gather(c, buf):
                return pltpu.make_async_copy(
                    tbl_sh.at[idx_v.at[c]], buf, gsem)

            def scatter(c, buf):
                return pltpu.make_async_copy(
                    buf, out_hbm.at[pl.ds(base + c * ch, ch)], ssem)

            for d in range(4):
                gather(d, bufs[d]).start()
            for d in range(8):
                gather(d, bufs[d % 8]).wait()
                scatter(d, bufs[d % 8]).start()
                if d >= 4:
                    scatter(d - 4, bufs[(d - 4) % 8]).wait()
                gather(d + 4, bufs[(d + 4) % 8]).start()

            @pl.loop(8, nch - 16, step=8)
            def _(c):
                for kk in range(8):
                    d = c + kk
                    gather(d, bufs[kk]).wait()
                    scatter(d, bufs[kk]).start()
                    scatter(d - 4, bufs[(kk + 4) % 8]).wait()
                    gather(d + 4, bufs[(kk + 4) % 8]).start()

            for dd in range(nch - 16, nch):
                gather(dd, bufs[dd % 8]).wait()
                scatter(dd, bufs[dd % 8]).start()
                scatter(dd - 4, bufs[(dd - 4) % 8]).wait()
                if dd + 4 < nch:
                    gather(dd + 4, bufs[(dd + 4) % 8]).start()

            for dd in range(nch - 4, nch):
                scatter(dd, bufs[dd % 8]).wait()

    return k(x, nlf)


def kernel(x, neighbor_list):
    b, n, xdim = x.shape
    l = neighbor_list.shape[-1]
    rows_per_tile = n * l // _NS
    assert b % _NC == 0 and (n * l) % _NS == 0 and rows_per_tile % _CH == 0
    assert n % _NS == 0

    nlf = neighbor_list.reshape(b * _NS, rows_per_tile // _CH, _CH)
    out = _gather_rows(x.reshape(b * n, xdim), nlf, b)
    return out.reshape(b, n, l, xdim)
